# Initial kernel scaffold; baseline (speedup 1.0000x reference)
import jax
import jax.numpy as jnp
from jax.experimental import pallas as pl


def kernel(x, edge_index, W0_1, W1_1, b1, W0_2, W1_2, b2):
    raise NotImplementedError("write your pallas kernel here")


# speedup vs baseline: 33.5622x; 33.5622x over previous
import functools

import jax
import jax.numpy as jnp
from jax import lax
from jax.experimental import pallas as pl
from jax.experimental.pallas import tpu as pltpu
from jax.experimental.pallas import tpu_sc as plsc

N = 10000
E = 320000
D = 128
H = 16

NC = 2
NS = 16
NW = NC * NS
EPW = E // NW
CHUNK = 128
KG = 8
NG = 10
NCHUNK = KG * NG
EPW_PAD = NCHUNK * CHUNK
ACC_ROWS = 10112
RPT = ACC_ROWS // NS

_mesh = plsc.VectorSubcoreMesh(core_axis_name="c", subcore_axis_name="s",
                               num_cores=NC, num_subcores=NS)


@functools.partial(
    pl.kernel,
    out_type=jax.ShapeDtypeStruct((NW, N), jnp.float32),
    mesh=_mesh,
    scratch_types=[
        pltpu.VMEM((EPW,), jnp.int32),
        pltpu.VMEM((N,), jnp.float32),
    ],
    compiler_params=pltpu.CompilerParams(needs_layout_passes=False),
)
def _deg_kernel(src_hbm, zeros_hbm, deg_out, src_v, deg_v):
    c = lax.axis_index("c")
    s = lax.axis_index("s")
    wid = s * NC + c
    pltpu.sync_copy(src_hbm.at[pl.ds(wid * EPW, EPW)], src_v)
    pltpu.sync_copy(zeros_hbm, deg_v)
    ones = jnp.ones((16,), jnp.float32)

    def body(i, carry):
        idx = src_v[pl.ds(i * 16, 16)]
        plsc.addupdate_scatter(deg_v, [idx], ones)
        return carry

    lax.fori_loop(0, EPW // 16, body, 0)
    pltpu.sync_copy(deg_v, deg_out.at[wid])


@functools.partial(
    pl.kernel,
    out_type=jax.ShapeDtypeStruct((NC, ACC_ROWS, H), jnp.float32),
    mesh=_mesh,
    scratch_types=[
        pltpu.VMEM((NCHUNK, CHUNK), jnp.int32),
        pltpu.VMEM((NCHUNK, CHUNK), jnp.int32),
        pltpu.VMEM((2, KG, CHUNK, H), jnp.float32),
        pltpu.VMEM_SHARED((ACC_ROWS, H), jnp.float32),
        pltpu.SemaphoreType.DMA,
        pltpu.SemaphoreType.DMA,
        pltpu.SemaphoreType.DMA,
        pltpu.SemaphoreType.DMA,
    ],
    compiler_params=pltpu.CompilerParams(needs_layout_passes=False,
                                         use_tc_tiling_on_sc=False),
)
def _agg_kernel(table_hbm, srcp_hbm, dstp_hbm, zacc_hbm, out_hbm,
                src_v, dst_v, buf_v, acc_sh, gsem0, gsem1, ssem0, ssem1):
    c = lax.axis_index("c")
    s = lax.axis_index("s")
    wid = s * NC + c
    pltpu.sync_copy(zacc_hbm.at[pl.ds(s * RPT, RPT)],
                    acc_sh.at[pl.ds(s * RPT, RPT)])
    pltpu.sync_copy(srcp_hbm.at[wid], src_v)
    pltpu.sync_copy(dstp_hbm.at[wid], dst_v)
    plsc.subcore_barrier()

    gsems = (gsem0, gsem1)
    ssems = (ssem0, ssem1)

    def fire_gathers(g):
        bs = g % 2
        return [
            pltpu.async_copy(table_hbm.at[src_v.at[g * KG + i]],
                             buf_v.at[bs, i], gsems[bs])
            for i in range(KG)
        ]

    def fire_scatters(g):
        bs = g % 2
        return [
            pltpu.async_copy(buf_v.at[bs, i],
                             acc_sh.at[dst_v.at[g * KG + i]], ssems[bs],
                             add=True)
            for i in range(KG)
        ]

    pend_g = {0: fire_gathers(0)}
    pend_s = {}
    for g in range(NG):
        for d in pend_g.pop(g):
            d.wait()
        pend_s[g] = fire_scatters(g)
        if g + 1 < NG:
            if g >= 1:
                for d in pend_s.pop(g - 1):
                    d.wait()
            pend_g[g + 1] = fire_gathers(g + 1)
    for g in sorted(pend_s):
        for d in pend_s[g]:
            d.wait()

    plsc.subcore_barrier()
    pltpu.sync_copy(acc_sh.at[pl.ds(s * RPT, RPT)],
                    out_hbm.at[c, pl.ds(s * RPT, RPT)])


def _dense1_body(x_ref, w0_ref, w1_ref, dp_ref, xw0_ref, xw1s_ref, dinv_ref):
    deg = jnp.sum(dp_ref[...], axis=0)
    dinv = jnp.where(deg > 0.0, lax.rsqrt(jnp.maximum(deg, 1e-12)), 0.0)
    dcol = dinv[:, None]
    xw0_ref[...] = jnp.dot(x_ref[...], w0_ref[...],
                           preferred_element_type=jnp.float32)
    xw1s_ref[...] = dcol * jnp.dot(x_ref[...], w1_ref[...],
                                   preferred_element_type=jnp.float32)
    dinv_ref[...] = dcol


_dense1 = pl.pallas_call(
    _dense1_body,
    out_shape=(
        jax.ShapeDtypeStruct((N, H), jnp.float32),
        jax.ShapeDtypeStruct((N, H), jnp.float32),
        jax.ShapeDtypeStruct((N, 1), jnp.float32),
    ),
)


def _dense2_body(xw0_ref, p_ref, dinv_ref, b1_ref, w0_ref, w1_ref,
                 hw0_ref, hw1s_ref):
    dcol = dinv_ref[...]
    t1 = -dcol * (p_ref[0] + p_ref[1])
    h = jnp.maximum(xw0_ref[...] + t1 + b1_ref[...][None, :], 0.0)
    hw0_ref[...] = jnp.dot(h, w0_ref[...], preferred_element_type=jnp.float32)
    hw1s_ref[...] = dcol * jnp.dot(h, w1_ref[...],
                                   preferred_element_type=jnp.float32)


_dense2 = pl.pallas_call(
    _dense2_body,
    out_shape=(
        jax.ShapeDtypeStruct((N, H), jnp.float32),
        jax.ShapeDtypeStruct((N, H), jnp.float32),
    ),
)


def _dense3_body(hw0_ref, q_ref, dinv_ref, b2_ref, out_ref):
    z = hw0_ref[...] - dinv_ref[...] * (q_ref[0] + q_ref[1]) \
        + b2_ref[...][None, :]
    m = jnp.max(z, axis=1, keepdims=True)
    lse = m + jnp.log(jnp.sum(jnp.exp(z - m), axis=1, keepdims=True))
    out_ref[...] = z - lse


_dense3 = pl.pallas_call(
    _dense3_body,
    out_shape=jax.ShapeDtypeStruct((N, H), jnp.float32),
)


def kernel(x, edge_index, W0_1, W1_1, b1, W0_2, W1_2, b2):
    src = edge_index[0]
    dst = edge_index[1]
    pad = EPW_PAD - EPW
    srcp = jnp.pad(src.reshape(NW, EPW), ((0, 0), (0, pad))) \
        .reshape(NW, NCHUNK, CHUNK)
    dstp = jnp.pad(dst.reshape(NW, EPW), ((0, 0), (0, pad)),
                   constant_values=N).reshape(NW, NCHUNK, CHUNK)
    zn = jnp.zeros((N,), jnp.float32)
    zacc = jnp.zeros((ACC_ROWS, H), jnp.float32)

    deg_part = _deg_kernel(src, zn)
    xw0, xw1s, dinv = _dense1(x, W0_1, W1_1, deg_part)
    p = _agg_kernel(xw1s, srcp, dstp, zacc)[:, :N, :]
    hw0, hw1s = _dense2(xw0, p, dinv, b1, W0_2, W1_2)
    q = _agg_kernel(hw1s, srcp, dstp, zacc)[:, :N, :]
    return _dense3(hw0, q, dinv, b2)


# speedup vs baseline: 47.8005x; 1.4242x over previous
import functools

import jax
import jax.numpy as jnp
from jax import lax
from jax.experimental import pallas as pl
from jax.experimental.pallas import tpu as pltpu
from jax.experimental.pallas import tpu_sc as plsc

N = 10000
E = 320000
D = 128
H = 16

NC = 2
NS = 16
NW = NC * NS
EPW = E // NW
CHUNK = 128
KG = 8
NG = 10
NCHUNK = KG * NG
EPW_PAD = NCHUNK * CHUNK
ACC_ROWS = 10112
RPT = ACC_ROWS // NS

_mesh = plsc.VectorSubcoreMesh(core_axis_name="c", subcore_axis_name="s",
                               num_cores=NC, num_subcores=NS)


@functools.partial(
    pl.kernel,
    out_type=jax.ShapeDtypeStruct((NW, N), jnp.float32),
    mesh=_mesh,
    scratch_types=[
        pltpu.VMEM((EPW,), jnp.int32),
        pltpu.VMEM((N,), jnp.float32),
    ],
    compiler_params=pltpu.CompilerParams(needs_layout_passes=False),
)
def _deg_kernel(src_hbm, zeros_hbm, deg_out, src_v, deg_v):
    c = lax.axis_index("c")
    s = lax.axis_index("s")
    wid = s * NC + c
    pltpu.sync_copy(src_hbm.at[pl.ds(wid * EPW, EPW)], src_v)
    pltpu.sync_copy(zeros_hbm, deg_v)
    ones = jnp.ones((16,), jnp.float32)

    def body(i, carry):
        idx = src_v[pl.ds(i * 16, 16)]
        plsc.addupdate_scatter(deg_v, [idx], ones)
        return carry

    lax.fori_loop(0, EPW // 16, body, 0)
    pltpu.sync_copy(deg_v, deg_out.at[wid])


@functools.partial(
    pl.kernel,
    out_type=jax.ShapeDtypeStruct((NC, ACC_ROWS, H), jnp.float32),
    mesh=_mesh,
    scratch_types=[
        pltpu.VMEM((NCHUNK, CHUNK), jnp.int32),
        pltpu.VMEM((NCHUNK, CHUNK), jnp.int32),
        pltpu.VMEM((2, KG, CHUNK, H), jnp.float32),
        pltpu.VMEM_SHARED((ACC_ROWS, H), jnp.float32),
        pltpu.VMEM_SHARED((N, H), jnp.float32),
        pltpu.SemaphoreType.DMA,
        pltpu.SemaphoreType.DMA,
        pltpu.SemaphoreType.DMA,
        pltpu.SemaphoreType.DMA,
    ],
    compiler_params=pltpu.CompilerParams(needs_layout_passes=False,
                                         use_tc_tiling_on_sc=False),
)
def _agg_kernel(table_hbm, srcp_hbm, dstp_hbm, zacc_hbm, out_hbm,
                src_v, dst_v, buf_v, acc_sh, tbl_sh,
                gsem0, gsem1, ssem0, ssem1):
    c = lax.axis_index("c")
    s = lax.axis_index("s")
    wid = s * NC + c
    pltpu.sync_copy(zacc_hbm.at[pl.ds(s * RPT, RPT)],
                    acc_sh.at[pl.ds(s * RPT, RPT)])
    pltpu.sync_copy(table_hbm.at[pl.ds(s * (N // NS), N // NS)],
                    tbl_sh.at[pl.ds(s * (N // NS), N // NS)])
    pltpu.sync_copy(srcp_hbm.at[wid], src_v)
    pltpu.sync_copy(dstp_hbm.at[wid], dst_v)
    plsc.subcore_barrier()

    gsems = (gsem0, gsem1)
    ssems = (ssem0, ssem1)

    def fire_gathers(g):
        bs = g % 2
        return [
            pltpu.async_copy(tbl_sh.at[src_v.at[g * KG + i]],
                             buf_v.at[bs, i], gsems[bs])
            for i in range(KG)
        ]

    def fire_scatters(g):
        bs = g % 2
        return [
            pltpu.async_copy(buf_v.at[bs, i],
                             acc_sh.at[dst_v.at[g * KG + i]], ssems[bs],
                             add=True)
            for i in range(KG)
        ]

    pend_g = {0: fire_gathers(0)}
    pend_s = {}
    for g in range(NG):
        for d in pend_g.pop(g):
            d.wait()
        pend_s[g] = fire_scatters(g)
        if g + 1 < NG:
            if g >= 1:
                for d in pend_s.pop(g - 1):
                    d.wait()
            pend_g[g + 1] = fire_gathers(g + 1)
    for g in sorted(pend_s):
        for d in pend_s[g]:
            d.wait()

    plsc.subcore_barrier()
    pltpu.sync_copy(acc_sh.at[pl.ds(s * RPT, RPT)],
                    out_hbm.at[c, pl.ds(s * RPT, RPT)])


def _dense1_body(x_ref, w0_ref, w1_ref, dp_ref, xw0_ref, xw1s_ref, dinv_ref):
    deg = jnp.sum(dp_ref[...], axis=0)
    dinv = jnp.where(deg > 0.0, lax.rsqrt(jnp.maximum(deg, 1e-12)), 0.0)
    dcol = dinv[:, None]
    xw0_ref[...] = jnp.dot(x_ref[...], w0_ref[...],
                           preferred_element_type=jnp.float32)
    xw1s_ref[...] = dcol * jnp.dot(x_ref[...], w1_ref[...],
                                   preferred_element_type=jnp.float32)
    dinv_ref[...] = dcol


_dense1 = pl.pallas_call(
    _dense1_body,
    out_shape=(
        jax.ShapeDtypeStruct((N, H), jnp.float32),
        jax.ShapeDtypeStruct((N, H), jnp.float32),
        jax.ShapeDtypeStruct((N, 1), jnp.float32),
    ),
)


def _dense2_body(xw0_ref, p_ref, dinv_ref, b1_ref, w0_ref, w1_ref,
                 hw0_ref, hw1s_ref):
    dcol = dinv_ref[...]
    t1 = -dcol * (p_ref[0] + p_ref[1])
    h = jnp.maximum(xw0_ref[...] + t1 + b1_ref[...][None, :], 0.0)
    hw0_ref[...] = jnp.dot(h, w0_ref[...], preferred_element_type=jnp.float32)
    hw1s_ref[...] = dcol * jnp.dot(h, w1_ref[...],
                                   preferred_element_type=jnp.float32)


_dense2 = pl.pallas_call(
    _dense2_body,
    out_shape=(
        jax.ShapeDtypeStruct((N, H), jnp.float32),
        jax.ShapeDtypeStruct((N, H), jnp.float32),
    ),
)


def _dense3_body(hw0_ref, q_ref, dinv_ref, b2_ref, out_ref):
    z = hw0_ref[...] - dinv_ref[...] * (q_ref[0] + q_ref[1]) \
        + b2_ref[...][None, :]
    m = jnp.max(z, axis=1, keepdims=True)
    lse = m + jnp.log(jnp.sum(jnp.exp(z - m), axis=1, keepdims=True))
    out_ref[...] = z - lse


_dense3 = pl.pallas_call(
    _dense3_body,
    out_shape=jax.ShapeDtypeStruct((N, H), jnp.float32),
)


def kernel(x, edge_index, W0_1, W1_1, b1, W0_2, W1_2, b2):
    src = edge_index[0]
    dst = edge_index[1]
    pad = EPW_PAD - EPW
    srcp = jnp.pad(src.reshape(NW, EPW), ((0, 0), (0, pad))) \
        .reshape(NW, NCHUNK, CHUNK)
    dstp = jnp.pad(dst.reshape(NW, EPW), ((0, 0), (0, pad)),
                   constant_values=N).reshape(NW, NCHUNK, CHUNK)
    zn = jnp.zeros((N,), jnp.float32)
    zacc = jnp.zeros((ACC_ROWS, H), jnp.float32)

    deg_part = _deg_kernel(src, zn)
    xw0, xw1s, dinv = _dense1(x, W0_1, W1_1, deg_part)
    p = _agg_kernel(xw1s, srcp, dstp, zacc)[:, :N, :]
    hw0, hw1s = _dense2(xw0, p, dinv, b1, W0_2, W1_2)
    q = _agg_kernel(hw1s, srcp, dstp, zacc)[:, :N, :]
    return _dense3(hw0, q, dinv, b2)


# speedup vs baseline: 49.1853x; 1.0290x over previous
import functools

import jax
import jax.numpy as jnp
from jax import lax
from jax.experimental import pallas as pl
from jax.experimental.pallas import tpu as pltpu
from jax.experimental.pallas import tpu_sc as plsc

N = 10000
E = 320000
D = 128
H = 16

NC = 2
NS = 16
NW = NC * NS
EPW = E // NW
CHUNK = 128
KG = 10
NG = 8
CPW = KG * NG
EPWA = CPW * CHUNK
EROWS = E // CHUNK
LW = NW - 1
NG_LAST = (E - LW * EPWA) // (KG * CHUNK)
ACC_ROWS = 10112
RPT = ACC_ROWS // NS
TPT = N // NS

_mesh = plsc.VectorSubcoreMesh(core_axis_name="c", subcore_axis_name="s",
                               num_cores=NC, num_subcores=NS)


@functools.partial(
    pl.kernel,
    out_type=jax.ShapeDtypeStruct((NW, N), jnp.float32),
    mesh=_mesh,
    scratch_types=[
        pltpu.VMEM((EPW,), jnp.int32),
        pltpu.VMEM((N,), jnp.float32),
    ],
    compiler_params=pltpu.CompilerParams(needs_layout_passes=False),
)
def _deg_kernel(src_hbm, zeros_hbm, deg_out, src_v, deg_v):
    c = lax.axis_index("c")
    s = lax.axis_index("s")
    wid = s * NC + c
    pltpu.sync_copy(src_hbm.at[pl.ds(wid * EPW, EPW)], src_v)
    pltpu.sync_copy(zeros_hbm, deg_v)
    ones = jnp.ones((16,), jnp.float32)

    def body(i, carry):
        for j in range(5):
            idx = src_v[pl.ds((i * 5 + j) * 16, 16)]
            plsc.addupdate_scatter(deg_v, [idx], ones)
        return carry

    lax.fori_loop(0, EPW // 80, body, 0)
    pltpu.sync_copy(deg_v, deg_out.at[wid])


@functools.partial(
    pl.kernel,
    out_type=jax.ShapeDtypeStruct((NC, ACC_ROWS, H), jnp.float32),
    mesh=_mesh,
    scratch_types=[
        pltpu.VMEM((EPWA,), jnp.int32),
        pltpu.VMEM((CPW, CHUNK), jnp.int32),
        pltpu.VMEM((2, KG, CHUNK, H), jnp.float32),
        pltpu.VMEM_SHARED((ACC_ROWS, H), jnp.float32),
        pltpu.VMEM_SHARED((N, H), jnp.float32),
        pltpu.SemaphoreType.DMA,
        pltpu.SemaphoreType.DMA,
        pltpu.SemaphoreType.DMA,
        pltpu.SemaphoreType.DMA,
    ],
    compiler_params=pltpu.CompilerParams(needs_layout_passes=False,
                                         use_tc_tiling_on_sc=False),
)
def _agg_kernel(table_hbm, src_hbm, dst2_hbm, zacc_hbm, out_hbm,
                src_v, dst_v, buf_v, acc_sh, tbl_sh,
                gsem0, gsem1, ssem0, ssem1):
    c = lax.axis_index("c")
    s = lax.axis_index("s")
    wid = s * NC + c
    pltpu.sync_copy(zacc_hbm.at[pl.ds(s * RPT, RPT)],
                    acc_sh.at[pl.ds(s * RPT, RPT)])
    pltpu.sync_copy(table_hbm.at[pl.ds(s * TPT, TPT)],
                    tbl_sh.at[pl.ds(s * TPT, TPT)])

    @pl.when(wid < LW)
    def _():
        pltpu.sync_copy(src_hbm.at[pl.ds(wid * EPWA, EPWA)], src_v)
        pltpu.sync_copy(dst2_hbm.at[pl.ds(wid * CPW, CPW)], dst_v)

    @pl.when(wid == LW)
    def _():
        pltpu.sync_copy(src_hbm.at[pl.ds(LW * EPWA, E - LW * EPWA)],
                        src_v.at[pl.ds(0, E - LW * EPWA)])
        pltpu.sync_copy(dst2_hbm.at[pl.ds(LW * CPW, EROWS - LW * CPW)],
                        dst_v.at[pl.ds(0, EROWS - LW * CPW)])

    plsc.subcore_barrier()

    my_ng = jnp.where(wid < LW, NG, NG_LAST)

    gsems = (gsem0, gsem1)
    ssems = (ssem0, ssem1)
    pend = {}

    def fire_gathers(g):
        bs = g % 2
        pend[('g', g)] = [
            pltpu.async_copy(
                tbl_sh.at[src_v.at[pl.ds((g * KG + i) * CHUNK, CHUNK)]],
                buf_v.at[bs, i], gsems[bs])
            for i in range(KG)
        ]

    def fire_scatters(g):
        bs = g % 2
        pend[('s', g)] = [
            pltpu.async_copy(buf_v.at[bs, i],
                             acc_sh.at[dst_v.at[g * KG + i]], ssems[bs],
                             add=True)
            for i in range(KG)
        ]

    def drain(kind, g):
        for d in pend.pop((kind, g)):
            d.wait()

    @pl.when(0 < my_ng)
    def _():
        fire_gathers(0)

    for g in range(NG):
        @pl.when(g < my_ng)
        def _(g=g):
            drain('g', g)
            fire_scatters(g)
        if g >= 1:
            @pl.when(g - 1 < my_ng)
            def _(g=g):
                drain('s', g - 1)
        if g + 1 < NG:
            @pl.when(g + 1 < my_ng)
            def _(g=g):
                fire_gathers(g + 1)

    @pl.when(NG - 1 < my_ng)
    def _():
        drain('s', NG - 1)

    plsc.subcore_barrier()
    pltpu.sync_copy(acc_sh.at[pl.ds(s * RPT, RPT)],
                    out_hbm.at[c, pl.ds(s * RPT, RPT)])


def _dense1_body(x_ref, w0_ref, w1_ref, dp_ref, xw0_ref, xw1s_ref, dinv_ref):
    deg = jnp.sum(dp_ref[...], axis=0)
    dinv = jnp.where(deg > 0.0, lax.rsqrt(jnp.maximum(deg, 1e-12)), 0.0)
    dcol = dinv[:, None]
    xw0_ref[...] = jnp.dot(x_ref[...], w0_ref[...],
                           preferred_element_type=jnp.float32)
    xw1s_ref[...] = dcol * jnp.dot(x_ref[...], w1_ref[...],
                                   preferred_element_type=jnp.float32)
    dinv_ref[...] = dcol


_dense1 = pl.pallas_call(
    _dense1_body,
    out_shape=(
        jax.ShapeDtypeStruct((N, H), jnp.float32),
        jax.ShapeDtypeStruct((N, H), jnp.float32),
        jax.ShapeDtypeStruct((N, 1), jnp.float32),
    ),
)


def _dense2_body(xw0_ref, p_ref, dinv_ref, b1_ref, w0_ref, w1_ref,
                 hw0_ref, hw1s_ref):
    dcol = dinv_ref[...]
    t1 = -dcol * (p_ref[0, :N, :] + p_ref[1, :N, :])
    h = jnp.maximum(xw0_ref[...] + t1 + b1_ref[...][None, :], 0.0)
    hw0_ref[...] = jnp.dot(h, w0_ref[...], preferred_element_type=jnp.float32)
    hw1s_ref[...] = dcol * jnp.dot(h, w1_ref[...],
                                   preferred_element_type=jnp.float32)


_dense2 = pl.pallas_call(
    _dense2_body,
    out_shape=(
        jax.ShapeDtypeStruct((N, H), jnp.float32),
        jax.ShapeDtypeStruct((N, H), jnp.float32),
    ),
)


def _dense3_body(hw0_ref, q_ref, dinv_ref, b2_ref, out_ref):
    z = hw0_ref[...] - dinv_ref[...] * (q_ref[0, :N, :] + q_ref[1, :N, :]) \
        + b2_ref[...][None, :]
    m = jnp.max(z, axis=1, keepdims=True)
    lse = m + jnp.log(jnp.sum(jnp.exp(z - m), axis=1, keepdims=True))
    out_ref[...] = z - lse


_dense3 = pl.pallas_call(
    _dense3_body,
    out_shape=jax.ShapeDtypeStruct((N, H), jnp.float32),
)


def kernel(x, edge_index, W0_1, W1_1, b1, W0_2, W1_2, b2):
    src = edge_index[0]
    dst2 = edge_index[1].reshape(EROWS, CHUNK)
    zn = jnp.zeros((N,), jnp.float32)
    zacc = jnp.zeros((ACC_ROWS, H), jnp.float32)

    deg_part = _deg_kernel(src, zn)
    xw0, xw1s, dinv = _dense1(x, W0_1, W1_1, deg_part)
    p = _agg_kernel(xw1s, src, dst2, zacc)
    hw0, hw1s = _dense2(xw0, p, dinv, b1, W0_2, W1_2)
    q = _agg_kernel(hw1s, src, dst2, zacc)
    return _dense3(hw0, q, dinv, b2)


# speedup vs baseline: 54.5395x; 1.1089x over previous
import functools

import jax
import jax.numpy as jnp
from jax import lax
from jax.experimental import pallas as pl
from jax.experimental.pallas import tpu as pltpu
from jax.experimental.pallas import tpu_sc as plsc

N = 10000
E = 320000
D = 128
H = 16

NC = 2
NS = 16
NW = NC * NS
EPW = E // NW
CHUNK = 128
KG = 10
NG = 8
CPW = KG * NG
EPWA = CPW * CHUNK
LW = NW - 1
NG_LAST = (E - LW * EPWA) // (KG * CHUNK)
ACC_ROWS = 10112
RPT = ACC_ROWS // NS
TPT = N // NS

_mesh = plsc.VectorSubcoreMesh(core_axis_name="c", subcore_axis_name="s",
                               num_cores=NC, num_subcores=NS)


@functools.partial(
    pl.kernel,
    out_type=jax.ShapeDtypeStruct((NW * N,), jnp.float32),
    mesh=_mesh,
    scratch_types=[
        pltpu.VMEM((EPW,), jnp.int32),
        pltpu.VMEM((N,), jnp.float32),
    ],
    compiler_params=pltpu.CompilerParams(needs_layout_passes=False,
                                         use_tc_tiling_on_sc=False),
)
def _deg_kernel(edge_hbm, zeros_hbm, deg_out, src_v, deg_v):
    c = lax.axis_index("c")
    s = lax.axis_index("s")
    wid = s * NC + c
    pltpu.sync_copy(edge_hbm.at[0, pl.ds(wid * EPW, EPW)], src_v)
    pltpu.sync_copy(zeros_hbm, deg_v)
    ones = jnp.ones((16,), jnp.float32)

    def body(i, carry):
        for j in range(5):
            idx = src_v[pl.ds((i * 5 + j) * 16, 16)]
            plsc.addupdate_scatter(deg_v, [idx], ones)
        return carry

    lax.fori_loop(0, EPW // 80, body, 0)
    pltpu.sync_copy(deg_v, deg_out.at[pl.ds(wid * N, N)])


@functools.partial(
    pl.kernel,
    out_type=jax.ShapeDtypeStruct((NC, ACC_ROWS, H), jnp.float32),
    mesh=_mesh,
    scratch_types=[
        pltpu.VMEM((EPWA,), jnp.int32),
        pltpu.VMEM((EPWA,), jnp.int32),
        pltpu.VMEM((2, KG, CHUNK, H), jnp.float32),
        pltpu.VMEM_SHARED((ACC_ROWS, H), jnp.float32),
        pltpu.VMEM_SHARED((N, H), jnp.float32),
        pltpu.SemaphoreType.DMA,
        pltpu.SemaphoreType.DMA,
        pltpu.SemaphoreType.DMA,
        pltpu.SemaphoreType.DMA,
    ],
    compiler_params=pltpu.CompilerParams(needs_layout_passes=False,
                                         use_tc_tiling_on_sc=False),
)
def _agg_kernel(table_hbm, edge_hbm, zacc_hbm, out_hbm,
                src_v, dst_v, buf_v, acc_sh, tbl_sh,
                gsem0, gsem1, ssem0, ssem1):
    c = lax.axis_index("c")
    s = lax.axis_index("s")
    wid = s * NC + c
    pltpu.sync_copy(zacc_hbm.at[pl.ds(s * RPT, RPT)],
                    acc_sh.at[pl.ds(s * RPT, RPT)])
    pltpu.sync_copy(table_hbm.at[pl.ds(s * TPT, TPT)],
                    tbl_sh.at[pl.ds(s * TPT, TPT)])

    @pl.when(wid < LW)
    def _():
        pltpu.sync_copy(edge_hbm.at[0, pl.ds(wid * EPWA, EPWA)], src_v)
        pltpu.sync_copy(edge_hbm.at[1, pl.ds(wid * EPWA, EPWA)], dst_v)

    @pl.when(wid == LW)
    def _():
        pltpu.sync_copy(edge_hbm.at[0, pl.ds(LW * EPWA, E - LW * EPWA)],
                        src_v.at[pl.ds(0, E - LW * EPWA)])
        pltpu.sync_copy(edge_hbm.at[1, pl.ds(LW * EPWA, E - LW * EPWA)],
                        dst_v.at[pl.ds(0, E - LW * EPWA)])

    plsc.subcore_barrier()

    my_ng = jnp.where(wid < LW, NG, NG_LAST)

    gsems = (gsem0, gsem1)
    ssems = (ssem0, ssem1)
    pend = {}

    def fire_gathers(g):
        bs = g % 2
        pend[('g', g)] = [
            pltpu.async_copy(
                tbl_sh.at[src_v.at[pl.ds((g * KG + i) * CHUNK, CHUNK)]],
                buf_v.at[bs, i], gsems[bs])
            for i in range(KG)
        ]

    def fire_scatters(g):
        bs = g % 2
        pend[('s', g)] = [
            pltpu.async_copy(
                buf_v.at[bs, i],
                acc_sh.at[dst_v.at[pl.ds((g * KG + i) * CHUNK, CHUNK)]],
                ssems[bs], add=True)
            for i in range(KG)
        ]

    def drain(kind, g):
        for d in pend.pop((kind, g)):
            d.wait()

    @pl.when(0 < my_ng)
    def _():
        fire_gathers(0)

    for g in range(NG):
        @pl.when(g < my_ng)
        def _(g=g):
            drain('g', g)
            fire_scatters(g)
        if g >= 1:
            @pl.when(g - 1 < my_ng)
            def _(g=g):
                drain('s', g - 1)
        if g + 1 < NG:
            @pl.when(g + 1 < my_ng)
            def _(g=g):
                fire_gathers(g + 1)

    @pl.when(NG - 1 < my_ng)
    def _():
        drain('s', NG - 1)

    plsc.subcore_barrier()
    pltpu.sync_copy(acc_sh.at[pl.ds(s * RPT, RPT)],
                    out_hbm.at[c, pl.ds(s * RPT, RPT)])


def _dinv_col(dp_ref):
    deg = dp_ref[pl.ds(0, N)]
    for w in range(1, NW):
        deg = deg + dp_ref[pl.ds(w * N, N)]
    dinv = jnp.where(deg > 0.0, lax.rsqrt(jnp.maximum(deg, 1e-12)), 0.0)
    return dinv[:, None]


def _dense1_body(x_ref, w0_ref, w1_ref, dp_ref, xw0_ref, xw1s_ref):
    dcol = _dinv_col(dp_ref)
    xw0_ref[...] = jnp.dot(x_ref[...], w0_ref[...],
                           preferred_element_type=jnp.float32)
    xw1s_ref[...] = dcol * jnp.dot(x_ref[...], w1_ref[...],
                                   preferred_element_type=jnp.float32)


_dense1 = pl.pallas_call(
    _dense1_body,
    out_shape=(
        jax.ShapeDtypeStruct((N, H), jnp.float32),
        jax.ShapeDtypeStruct((N, H), jnp.float32),
    ),
)


def _dense2_body(xw0_ref, p_ref, dp_ref, b1_ref, w0_ref, w1_ref,
                 hw0_ref, hw1s_ref):
    dcol = _dinv_col(dp_ref)
    t1 = -dcol * (p_ref[0, :N, :] + p_ref[1, :N, :])
    h = jnp.maximum(xw0_ref[...] + t1 + b1_ref[...][None, :], 0.0)
    hw0_ref[...] = jnp.dot(h, w0_ref[...], preferred_element_type=jnp.float32)
    hw1s_ref[...] = dcol * jnp.dot(h, w1_ref[...],
                                   preferred_element_type=jnp.float32)


_dense2 = pl.pallas_call(
    _dense2_body,
    out_shape=(
        jax.ShapeDtypeStruct((N, H), jnp.float32),
        jax.ShapeDtypeStruct((N, H), jnp.float32),
    ),
)


def _dense3_body(hw0_ref, q_ref, dp_ref, b2_ref, out_ref):
    z = hw0_ref[...] - _dinv_col(dp_ref) * (q_ref[0, :N, :] + q_ref[1, :N, :]) \
        + b2_ref[...][None, :]
    m = jnp.max(z, axis=1, keepdims=True)
    lse = m + jnp.log(jnp.sum(jnp.exp(z - m), axis=1, keepdims=True))
    out_ref[...] = z - lse


_dense3 = pl.pallas_call(
    _dense3_body,
    out_shape=jax.ShapeDtypeStruct((N, H), jnp.float32),
)


def kernel(x, edge_index, W0_1, W1_1, b1, W0_2, W1_2, b2):
    zn = jnp.zeros((N,), jnp.float32)
    zacc = jnp.zeros((ACC_ROWS, H), jnp.float32)

    deg_part = _deg_kernel(edge_index, zn)
    xw0, xw1s = _dense1(x, W0_1, W1_1, deg_part)
    p = _agg_kernel(xw1s, edge_index, zacc)
    hw0, hw1s = _dense2(xw0, p, deg_part, b1, W0_2, W1_2)
    q = _agg_kernel(hw1s, edge_index, zacc)
    return _dense3(hw0, q, deg_part, b2)
